# Initial kernel scaffold; baseline (speedup 1.0000x reference)
import jax
import jax.numpy as jnp
from jax.experimental import pallas as pl


def kernel(features, h3_nodes, in_edge_index, in_edge_attr, lat_edge_index, lat_edge_attr, params):
    raise NotImplementedError("write your pallas kernel here")


# speedup vs baseline: 2.1227x; 2.1227x over previous
import dataclasses
import functools

import jax
import jax.numpy as jnp
from jax import lax
from jax.experimental import pallas as pl
from jax.experimental.pallas import tpu as pltpu
from jax.experimental.pallas import tpu_sc as plsc

F32 = jnp.float32
BF16 = jnp.bfloat16

D = 256
NW = 32
EW = 128
CHUNKS = 13
CHUNKS_PAD = 16
PE = NW * CHUNKS * EW
HALF = PE // 2
CH = 1024
P3 = 5888
PL = 41472
BM = 512
BM3 = 1472

_CP = pltpu.CompilerParams()
if "needs_layout_passes" in pltpu.CompilerParams.__dataclass_fields__:
    _CP = dataclasses.replace(_CP, needs_layout_passes=False)


def _ln(x, g, b):
    mu = jnp.mean(x, axis=-1, keepdims=True)
    xc = x - mu
    var = jnp.mean(xc * xc, axis=-1, keepdims=True)
    return xc * lax.rsqrt(var + 1e-5) * g + b


def _bdot(a, w_ref):
    return jnp.dot(a.astype(BF16), w_ref[...], preferred_element_type=F32)


def _enc_small_k(inp, w0_ref, b0_ref):
    k = inp.shape[-1]
    acc = b0_ref[...]
    for j in range(k):
        acc = acc + inp[:, j : j + 1] * w0_ref[j : j + 1, :]
    return jnp.maximum(acc, 0.0)


def _mlp3(inp, refs):
    w0, b0, w1, b1, w2, b2, g, beta = refs
    h = _enc_small_k(inp, w0, b0)
    h = jnp.maximum(_bdot(h, w1) + b1[...], 0.0)
    return _ln(_bdot(h, w2) + b2[...], g[...], beta[...])


def _full(a):
    return pl.BlockSpec(a.shape, lambda *_: (0,) * a.ndim)


def _rows(bm, nc):
    return pl.BlockSpec((bm, nc), lambda i: (i, 0))


def _h3_prep_kernel(h3_ref, w0, b0, w1, b1, w2, b2, g, beta, p1b, x_out, y_out):
    x = _mlp3(h3_ref[...], (w0, b0, w1, b1, w2, b2, g, beta))
    x_out[...] = x
    y_out[...] = _bdot(x, p1b)


def _h3_prep(h3p, ne, p1b):
    grid = (P3 // BM3,)
    args = (h3p, *ne, p1b)
    in_specs = [_rows(BM3, h3p.shape[1])] + [_full(a) for a in args[1:]]
    return pl.pallas_call(
        _h3_prep_kernel,
        grid=grid,
        in_specs=in_specs,
        out_specs=[_rows(BM3, D), _rows(BM3, D)],
        out_shape=[
            jax.ShapeDtypeStruct((P3, D), F32),
            jax.ShapeDtypeStruct((P3, D), F32),
        ],
    )(*args)


def _edges_kernel(feat_ref, attr_ref, ydst_ref,
                  nw0, nb0, nw1, nb1, nw2, nb2, ng, nbeta,
                  ew0, eb0, ew1, eb1, ew2, eb2, eg, ebeta,
                  p1a, p1c, pb1, p2, pb2, p3w, pb3, pg, pbeta,
                  out_ref):
    x = _mlp3(feat_ref[...], (nw0, nb0, nw1, nb1, nw2, nb2, ng, nbeta))
    e = _mlp3(attr_ref[...], (ew0, eb0, ew1, eb1, ew2, eb2, eg, ebeta))
    h = jnp.maximum(_bdot(x, p1a) + _bdot(e, p1c) + ydst_ref[...] + pb1[...], 0.0)
    h = jnp.maximum(_bdot(h, p2) + pb2[...], 0.0)
    e_upd = _ln(_bdot(h, p3w) + pb3[...], pg[...], pbeta[...]) + e
    out_ref[...] = e_upd.T


def _edges(featp, attrp, y_dst, ne, ee, pw):
    grid = (PE // BM,)
    args = (featp, attrp, y_dst, *ne, *ee, *pw)
    in_specs = [
        _rows(BM, featp.shape[1]),
        _rows(BM, attrp.shape[1]),
        _rows(BM, D),
    ] + [_full(a) for a in args[3:]]
    return pl.pallas_call(
        _edges_kernel,
        grid=grid,
        in_specs=in_specs,
        out_specs=pl.BlockSpec((D, BM), lambda i: (0, i)),
        out_shape=jax.ShapeDtypeStruct((D, PE), F32),
    )(*args)


def _lat_kernel(attr_ref, w0, b0, w1, b1, w2, b2, g, beta, out_ref):
    out_ref[...] = _mlp3(attr_ref[...], (w0, b0, w1, b1, w2, b2, g, beta))


def _lat(latp, le):
    grid = (PL // BM,)
    args = (latp, *le)
    in_specs = [_rows(BM, latp.shape[1])] + [_full(a) for a in args[1:]]
    return pl.pallas_call(
        _lat_kernel,
        grid=grid,
        in_specs=in_specs,
        out_specs=_rows(BM, D),
        out_shape=jax.ShapeDtypeStruct((PL, D), F32),
    )(*args)


def _nodeproc_kernel(x_ref, aggt_ref,
                     m1a, m1b, mb1, m2, mb2, m3, mb3, mg, mbeta,
                     out_ref):
    x = x_ref[...]
    aggt = aggt_ref[...]
    agg = (aggt[0] + aggt[1]).T
    h = jnp.maximum(_bdot(x, m1a) + _bdot(agg, m1b) + mb1[...], 0.0)
    h = jnp.maximum(_bdot(h, m2) + mb2[...], 0.0)
    out_ref[...] = _ln(_bdot(h, m3) + mb3[...], mg[...], mbeta[...]) + x


def _nodeproc(x_h3, aggT, nw):
    bm = 256
    grid = (P3 // bm,)
    args = (x_h3, aggT, *nw)
    in_specs = [
        _rows(bm, D),
        pl.BlockSpec((2, D, bm), lambda i: (0, 0, i)),
    ] + [_full(a) for a in args[2:]]
    return pl.pallas_call(
        _nodeproc_kernel,
        grid=grid,
        in_specs=in_specs,
        out_specs=_rows(bm, D),
        out_shape=jax.ShapeDtypeStruct((P3, D), F32),
    )(*args)


def _sc_gather(table, idx2d):
    mesh = plsc.VectorSubcoreMesh(core_axis_name="core", subcore_axis_name="subcore")

    @functools.partial(
        pl.kernel,
        out_type=jax.ShapeDtypeStruct((PE, D), F32),
        mesh=mesh,
        scratch_types=[
            pltpu.VMEM((CHUNKS_PAD, EW), jnp.int32),
            pltpu.VMEM((EW, D), F32),
            pltpu.SemaphoreType.DMA,
        ],
    )
    def k(table_hbm, i_hbm, o_hbm, idx_v, rows_v, sem):
        wid = lax.axis_index("subcore") * 2 + lax.axis_index("core")
        pltpu.sync_copy(i_hbm.at[wid], idx_v)
        base = wid * CHUNKS * EW

        @pl.loop(0, CHUNKS)
        def _(j):
            pltpu.async_copy(table_hbm.at[idx_v.at[j]], rows_v, sem).wait()
            pltpu.sync_copy(rows_v, o_hbm.at[pl.ds(base + j * EW, EW)])

    return k(table, idx2d)


def _sc_scatter(e_updT, idx1d, zerosT):
    mesh = plsc.VectorSubcoreMesh(core_axis_name="core", subcore_axis_name="subcore")

    @functools.partial(
        pl.kernel,
        out_type=jax.ShapeDtypeStruct((2, D, P3), F32),
        mesh=mesh,
        scratch_types=[
            pltpu.VMEM((16, P3), F32),
            pltpu.VMEM((CH,), jnp.int32),
            pltpu.VMEM((16, CH), F32),
        ],
        compiler_params=_CP,
    )
    def k(x_hbm, i_hbm, z_hbm, o_hbm, acc, idx_v, x_v):
        c = lax.axis_index("core")
        s = lax.axis_index("subcore")
        pltpu.sync_copy(z_hbm, acc)
        col0 = s * 16
        ebase = c * HALF

        @pl.loop(0, HALF // CH)
        def _(t):
            off = ebase + t * CH
            pltpu.sync_copy(i_hbm.at[pl.ds(off, CH)], idx_v)
            pltpu.sync_copy(x_hbm.at[pl.ds(col0, 16), pl.ds(off, CH)], x_v)

            @pl.loop(0, CH // 16)
            def _(g):
                v_idx = idx_v[pl.ds(g * 16, 16)]
                for kk in range(16):
                    rowk = jnp.full((16,), kk, jnp.int32)
                    plsc.addupdate_scatter(
                        acc, [rowk, v_idx], x_v[kk, pl.ds(g * 16, 16)]
                    )

        pltpu.sync_copy(acc, o_hbm.at[c, pl.ds(col0, 16)])

    return k(e_updT, idx1d, zerosT)


def _prep_mlp(p):
    (w0, b0), (w1, b1), (w2, b2) = p["layers"]
    r = lambda v: v.reshape(1, -1)
    return (
        w0,
        r(b0),
        w1.astype(BF16),
        r(b1),
        w2.astype(BF16),
        r(b2),
        r(p["ln_g"]),
        r(p["ln_b"]),
    )


def kernel(features, h3_nodes, in_edge_index, in_edge_attr, lat_edge_index, lat_edge_attr, params):
    n_obs = features.shape[1]
    n3 = h3_nodes.shape[0]
    n_lat = lat_edge_attr.shape[0]

    ne = _prep_mlp(params["node_encoder"])
    ee = _prep_mlp(params["edge_encoder"])
    le = _prep_mlp(params["latent_edge_encoder"])

    pe = params["edge_proc"]
    p1 = pe["layers"][0][0]
    p1a, p1b, p1c = p1[:D].astype(BF16), p1[D : 2 * D].astype(BF16), p1[2 * D :].astype(BF16)
    r = lambda v: v.reshape(1, -1)
    pw = (
        p1a, p1c, r(pe["layers"][0][1]),
        pe["layers"][1][0].astype(BF16), r(pe["layers"][1][1]),
        pe["layers"][2][0].astype(BF16), r(pe["layers"][2][1]),
        r(pe["ln_g"]), r(pe["ln_b"]),
    )

    pn = params["node_proc"]
    m1 = pn["layers"][0][0]
    nw = (
        m1[:D].astype(BF16), m1[D:].astype(BF16), r(pn["layers"][0][1]),
        pn["layers"][1][0].astype(BF16), r(pn["layers"][1][1]),
        pn["layers"][2][0].astype(BF16), r(pn["layers"][2][1]),
        r(pn["ln_g"]), r(pn["ln_b"]),
    )

    featp = jnp.pad(features.reshape(-1, features.shape[-1]), ((0, PE - n_obs), (0, 0)))
    attrp = jnp.pad(in_edge_attr, ((0, PE - n_obs), (0, 0)))
    h3p = jnp.pad(h3_nodes, ((0, P3 - n3), (0, 0)))
    latp = jnp.pad(lat_edge_attr, ((0, PL - n_lat), (0, 0)))
    idx = in_edge_index[1] - n_obs
    pad_tail = n3 + jnp.arange(PE - n_obs, dtype=jnp.int32) % (P3 - n3)
    idxp = jnp.concatenate([idx, pad_tail])
    idx3d_g = jnp.pad(
        idxp.reshape(NW, CHUNKS, EW),
        ((0, 0), (0, CHUNKS_PAD - CHUNKS), (0, 0)),
        constant_values=n3,
    )
    zerosT = jnp.zeros((16, P3), F32)

    x_h3, y_h3 = _h3_prep(h3p, ne, p1b)
    y_dst = _sc_gather(y_h3, idx3d_g)
    e_updT = _edges(featp, attrp, y_dst, ne, ee, pw)
    aggT = _sc_scatter(e_updT, idxp, zerosT)
    lat_e = _lat(latp, le)
    out = _nodeproc(x_h3, aggT, nw)

    return out[:n3], lat_edge_index, lat_e[:n_lat]


# speedup vs baseline: 2.3462x; 1.1053x over previous
import dataclasses
import functools

import jax
import jax.numpy as jnp
from jax import lax
from jax.experimental import pallas as pl
from jax.experimental.pallas import tpu as pltpu
from jax.experimental.pallas import tpu_sc as plsc

F32 = jnp.float32
BF16 = jnp.bfloat16

D = 256
NW = 32
EW = 128
CHUNKS = 13
CHUNKS_PAD = 16
PE = NW * CHUNKS * EW
HALF = PE // 2
CH = 1024
P3 = 5888
PL = 41472
BM = 512
BM3 = 1472

_CP = pltpu.CompilerParams()
if "needs_layout_passes" in pltpu.CompilerParams.__dataclass_fields__:
    _CP = dataclasses.replace(_CP, needs_layout_passes=False)


def _ln(x, g, b):
    mu = jnp.mean(x, axis=-1, keepdims=True)
    xc = x - mu
    var = jnp.mean(xc * xc, axis=-1, keepdims=True)
    return xc * lax.rsqrt(var + 1e-5) * g + b


def _bdot(a, w_ref):
    return jnp.dot(a.astype(BF16), w_ref[...], preferred_element_type=F32)


def _enc_small_k(inp, w0_ref, b0_ref):
    k = inp.shape[-1]
    acc = b0_ref[...]
    for j in range(k):
        acc = acc + inp[:, j : j + 1] * w0_ref[j : j + 1, :]
    return jnp.maximum(acc, 0.0)


def _mlp3(inp, refs):
    w0, b0, w1, b1, w2, b2, g, beta = refs
    h = _enc_small_k(inp, w0, b0)
    h = jnp.maximum(_bdot(h, w1) + b1[...], 0.0)
    return _ln(_bdot(h, w2) + b2[...], g[...], beta[...])


def _full(a):
    return pl.BlockSpec(a.shape, lambda *_: (0,) * a.ndim)


def _rows(bm, nc):
    return pl.BlockSpec((bm, nc), lambda i: (i, 0))


def _h3_prep_kernel(h3_ref, w0, b0, w1, b1, w2, b2, g, beta, p1b, x_out, y_out):
    x = _mlp3(h3_ref[...], (w0, b0, w1, b1, w2, b2, g, beta))
    x_out[...] = x
    y_out[...] = _bdot(x, p1b)


def _h3_prep(h3p, ne, p1b):
    grid = (P3 // BM3,)
    args = (h3p, *ne, p1b)
    in_specs = [_rows(BM3, h3p.shape[1])] + [_full(a) for a in args[1:]]
    return pl.pallas_call(
        _h3_prep_kernel,
        grid=grid,
        in_specs=in_specs,
        out_specs=[_rows(BM3, D), _rows(BM3, D)],
        out_shape=[
            jax.ShapeDtypeStruct((P3, D), F32),
            jax.ShapeDtypeStruct((P3, D), F32),
        ],
    )(*args)


def _edges_kernel(feat_ref, attr_ref, ydst_ref,
                  nw0, nb0, nw1, nb1, nw2, nb2, ng, nbeta,
                  ew0, eb0, ew1, eb1, ew2, eb2, eg, ebeta,
                  p1a, p1c, pb1, p2, pb2, p3w, pb3, pg, pbeta,
                  out_ref):
    x = _mlp3(feat_ref[...], (nw0, nb0, nw1, nb1, nw2, nb2, ng, nbeta))
    e = _mlp3(attr_ref[...], (ew0, eb0, ew1, eb1, ew2, eb2, eg, ebeta))
    h = jnp.maximum(_bdot(x, p1a) + _bdot(e, p1c) + ydst_ref[...] + pb1[...], 0.0)
    h = jnp.maximum(_bdot(h, p2) + pb2[...], 0.0)
    e_upd = _ln(_bdot(h, p3w) + pb3[...], pg[...], pbeta[...]) + e
    out_ref[...] = e_upd.T


def _edges(featp, attrp, y_dst, ne, ee, pw):
    grid = (PE // BM,)
    args = (featp, attrp, y_dst, *ne, *ee, *pw)
    in_specs = [
        _rows(BM, featp.shape[1]),
        _rows(BM, attrp.shape[1]),
        _rows(BM, D),
    ] + [_full(a) for a in args[3:]]
    return pl.pallas_call(
        _edges_kernel,
        grid=grid,
        in_specs=in_specs,
        out_specs=pl.BlockSpec((D, BM), lambda i: (0, i)),
        out_shape=jax.ShapeDtypeStruct((D, PE), F32),
    )(*args)


def _lat_kernel(attr_ref, w0, b0, w1, b1, w2, b2, g, beta, out_ref):
    out_ref[...] = _mlp3(attr_ref[...], (w0, b0, w1, b1, w2, b2, g, beta))


def _lat(latp, le):
    grid = (PL // BM,)
    args = (latp, *le)
    in_specs = [_rows(BM, latp.shape[1])] + [_full(a) for a in args[1:]]
    return pl.pallas_call(
        _lat_kernel,
        grid=grid,
        in_specs=in_specs,
        out_specs=_rows(BM, D),
        out_shape=jax.ShapeDtypeStruct((PL, D), F32),
    )(*args)


def _nodeproc_kernel(x_ref, aggt_ref,
                     m1a, m1b, mb1, m2, mb2, m3, mb3, mg, mbeta,
                     out_ref):
    x = x_ref[...]
    aggt = aggt_ref[...]
    agg = (aggt[0] + aggt[1]).T
    h = jnp.maximum(_bdot(x, m1a) + _bdot(agg, m1b) + mb1[...], 0.0)
    h = jnp.maximum(_bdot(h, m2) + mb2[...], 0.0)
    out_ref[...] = _ln(_bdot(h, m3) + mb3[...], mg[...], mbeta[...]) + x


def _nodeproc(x_h3, aggT, nw):
    bm = 256
    grid = (P3 // bm,)
    args = (x_h3, aggT, *nw)
    in_specs = [
        _rows(bm, D),
        pl.BlockSpec((2, D, bm), lambda i: (0, 0, i)),
    ] + [_full(a) for a in args[2:]]
    return pl.pallas_call(
        _nodeproc_kernel,
        grid=grid,
        in_specs=in_specs,
        out_specs=_rows(bm, D),
        out_shape=jax.ShapeDtypeStruct((P3, D), F32),
    )(*args)


def _sc_gather(table, idx2d):
    mesh = plsc.VectorSubcoreMesh(core_axis_name="core", subcore_axis_name="subcore")

    @functools.partial(
        pl.kernel,
        out_type=jax.ShapeDtypeStruct((PE, D), F32),
        mesh=mesh,
        scratch_types=[
            pltpu.VMEM((CHUNKS_PAD, EW), jnp.int32),
            pltpu.VMEM((EW, D), F32),
            pltpu.VMEM((EW, D), F32),
            pltpu.SemaphoreType.DMA,
            pltpu.SemaphoreType.DMA,
        ],
    )
    def k(table_hbm, i_hbm, o_hbm, idx_v, rows0, rows1, sem0, sem1):
        wid = lax.axis_index("subcore") * 2 + lax.axis_index("core")
        pltpu.sync_copy(i_hbm.at[wid], idx_v)
        base = wid * CHUNKS * EW
        bufs = ((rows0, sem0), (rows1, sem1))

        def start(b, j):
            pltpu.async_copy(table_hbm.at[idx_v.at[j]], bufs[b][0], bufs[b][1])

        def finish(b, j):
            pltpu.make_async_copy(table_hbm.at[idx_v.at[j]], bufs[b][0], bufs[b][1]).wait()
            pltpu.sync_copy(bufs[b][0], o_hbm.at[pl.ds(base + j * EW, EW)])

        start(0, 0)

        @pl.loop(0, CHUNKS // 2)
        def _(p):
            j0 = p * 2
            start(1, j0 + 1)
            finish(0, j0)

            @pl.when(j0 + 2 < CHUNKS)
            def _():
                start(0, j0 + 2)

            finish(1, j0 + 1)

        finish(0, CHUNKS - 1)

    return k(table, idx2d)


def _sc_scatter(e_updT, idx1d, zerosT):
    mesh = plsc.VectorSubcoreMesh(core_axis_name="core", subcore_axis_name="subcore")

    @functools.partial(
        pl.kernel,
        out_type=jax.ShapeDtypeStruct((2, D, P3), F32),
        mesh=mesh,
        scratch_types=[
            pltpu.VMEM((16, P3), F32),
            pltpu.VMEM((CH,), jnp.int32),
            pltpu.VMEM((CH,), jnp.int32),
            pltpu.VMEM((16, CH), F32),
            pltpu.VMEM((16, CH), F32),
            pltpu.SemaphoreType.DMA,
            pltpu.SemaphoreType.DMA,
        ],
        compiler_params=_CP,
    )
    def k(x_hbm, i_hbm, z_hbm, o_hbm, acc, idx0, idx1, x0, x1, sem0, sem1):
        c = lax.axis_index("core")
        s = lax.axis_index("subcore")
        pltpu.sync_copy(z_hbm, acc)
        col0 = s * 16
        ebase = c * HALF
        nch = HALF // CH
        bufs = ((idx0, x0, sem0), (idx1, x1, sem1))

        def start(b, t):
            off = ebase + t * CH
            iv, xv, sem = bufs[b]
            pltpu.async_copy(i_hbm.at[pl.ds(off, CH)], iv, sem)
            pltpu.async_copy(x_hbm.at[pl.ds(col0, 16), pl.ds(off, CH)], xv, sem)

        def compute(b, t):
            off = ebase + t * CH
            iv, xv, sem = bufs[b]
            pltpu.make_async_copy(i_hbm.at[pl.ds(off, CH)], iv, sem).wait()
            pltpu.make_async_copy(
                x_hbm.at[pl.ds(col0, 16), pl.ds(off, CH)], xv, sem
            ).wait()

            @pl.loop(0, CH // 16)
            def _(g):
                v_idx = iv[pl.ds(g * 16, 16)]
                for kk in range(16):
                    rowk = jnp.full((16,), kk, jnp.int32)
                    plsc.addupdate_scatter(
                        acc, [rowk, v_idx], xv[kk, pl.ds(g * 16, 16)]
                    )

        start(0, 0)

        @pl.loop(0, nch // 2)
        def _(p):
            t0 = p * 2
            start(1, t0 + 1)
            compute(0, t0)

            @pl.when(t0 + 2 < nch)
            def _():
                start(0, t0 + 2)

            compute(1, t0 + 1)

        pltpu.sync_copy(acc, o_hbm.at[c, pl.ds(col0, 16)])

    return k(e_updT, idx1d, zerosT)


def _prep_mlp(p):
    (w0, b0), (w1, b1), (w2, b2) = p["layers"]
    r = lambda v: v.reshape(1, -1)
    return (
        w0,
        r(b0),
        w1.astype(BF16),
        r(b1),
        w2.astype(BF16),
        r(b2),
        r(p["ln_g"]),
        r(p["ln_b"]),
    )


def kernel(features, h3_nodes, in_edge_index, in_edge_attr, lat_edge_index, lat_edge_attr, params):
    n_obs = features.shape[1]
    n3 = h3_nodes.shape[0]
    n_lat = lat_edge_attr.shape[0]

    ne = _prep_mlp(params["node_encoder"])
    ee = _prep_mlp(params["edge_encoder"])
    le = _prep_mlp(params["latent_edge_encoder"])

    pe = params["edge_proc"]
    p1 = pe["layers"][0][0]
    p1a, p1b, p1c = p1[:D].astype(BF16), p1[D : 2 * D].astype(BF16), p1[2 * D :].astype(BF16)
    r = lambda v: v.reshape(1, -1)
    pw = (
        p1a, p1c, r(pe["layers"][0][1]),
        pe["layers"][1][0].astype(BF16), r(pe["layers"][1][1]),
        pe["layers"][2][0].astype(BF16), r(pe["layers"][2][1]),
        r(pe["ln_g"]), r(pe["ln_b"]),
    )

    pn = params["node_proc"]
    m1 = pn["layers"][0][0]
    nw = (
        m1[:D].astype(BF16), m1[D:].astype(BF16), r(pn["layers"][0][1]),
        pn["layers"][1][0].astype(BF16), r(pn["layers"][1][1]),
        pn["layers"][2][0].astype(BF16), r(pn["layers"][2][1]),
        r(pn["ln_g"]), r(pn["ln_b"]),
    )

    featp = jnp.pad(features.reshape(-1, features.shape[-1]), ((0, PE - n_obs), (0, 0)))
    attrp = jnp.pad(in_edge_attr, ((0, PE - n_obs), (0, 0)))
    h3p = jnp.pad(h3_nodes, ((0, P3 - n3), (0, 0)))
    latp = jnp.pad(lat_edge_attr, ((0, PL - n_lat), (0, 0)))
    idx = in_edge_index[1] - n_obs
    pad_tail = n3 + jnp.arange(PE - n_obs, dtype=jnp.int32) % (P3 - n3)
    idxp = jnp.concatenate([idx, pad_tail])
    idx3d_g = jnp.pad(
        idxp.reshape(NW, CHUNKS, EW),
        ((0, 0), (0, CHUNKS_PAD - CHUNKS), (0, 0)),
        constant_values=n3,
    )
    zerosT = jnp.zeros((16, P3), F32)

    x_h3, y_h3 = _h3_prep(h3p, ne, p1b)
    y_dst = _sc_gather(y_h3, idx3d_g)
    e_updT = _edges(featp, attrp, y_dst, ne, ee, pw)
    aggT = _sc_scatter(e_updT, idxp, zerosT)
    lat_e = _lat(latp, le)
    out = _nodeproc(x_h3, aggT, nw)

    return out[:n3], lat_edge_index, lat_e[:n_lat]


# speedup vs baseline: 2.4300x; 1.0357x over previous
import dataclasses
import functools

import jax
import jax.numpy as jnp
from jax import lax
from jax.experimental import pallas as pl
from jax.experimental.pallas import tpu as pltpu
from jax.experimental.pallas import tpu_sc as plsc

F32 = jnp.float32
BF16 = jnp.bfloat16

D = 256
NW = 32
EW = 128
CHUNKS = 13
CHUNKS_PAD = 16
PE = NW * CHUNKS * EW
HALF = PE // 2
CH = 1024
P3 = 5888
PL = 41472
BM = 512
BM3 = 1472

_CP = pltpu.CompilerParams()
if "needs_layout_passes" in pltpu.CompilerParams.__dataclass_fields__:
    _CP = dataclasses.replace(_CP, needs_layout_passes=False)


def _ln(x, g, b):
    mu = jnp.mean(x, axis=-1, keepdims=True)
    xc = x - mu
    var = jnp.mean(xc * xc, axis=-1, keepdims=True)
    return xc * lax.rsqrt(var + 1e-5) * g + b


def _bdot(a, w_ref):
    return jnp.dot(a, w_ref[...], preferred_element_type=F32)


def _enc_small_k(inp, w0_ref, b0_ref):
    return jnp.maximum(_bdot(inp, w0_ref) + b0_ref[...], 0.0)


def _mlp3(inp, refs):
    w0, b0, w1, b1, w2, b2, g, beta = refs
    h = _enc_small_k(inp, w0, b0)
    h = jnp.maximum(_bdot(h, w1) + b1[...], 0.0)
    return _ln(_bdot(h, w2) + b2[...], g[...], beta[...])


def _full(a):
    return pl.BlockSpec(a.shape, lambda *_: (0,) * a.ndim)


def _rows(bm, nc):
    return pl.BlockSpec((bm, nc), lambda i: (i, 0))


def _h3_prep_kernel(h3_ref, w0, b0, w1, b1, w2, b2, g, beta, p1b, x_out, y_out):
    x = _mlp3(h3_ref[...], (w0, b0, w1, b1, w2, b2, g, beta))
    x_out[...] = x
    y_out[...] = _bdot(x, p1b)


def _h3_prep(h3p, ne, p1b):
    grid = (P3 // BM3,)
    args = (h3p, *ne, p1b)
    in_specs = [_rows(BM3, h3p.shape[1])] + [_full(a) for a in args[1:]]
    return pl.pallas_call(
        _h3_prep_kernel,
        grid=grid,
        in_specs=in_specs,
        out_specs=[_rows(BM3, D), _rows(BM3, D)],
        out_shape=[
            jax.ShapeDtypeStruct((P3, D), F32),
            jax.ShapeDtypeStruct((P3, D), F32),
        ],
    )(*args)


def _edges_kernel(feat_ref, attr_ref, ydst_ref,
                  nw0, nb0, nw1, nb1, nw2, nb2, ng, nbeta,
                  ew0, eb0, ew1, eb1, ew2, eb2, eg, ebeta,
                  p1a, p1c, pb1, p2, pb2, p3w, pb3, pg, pbeta,
                  out_ref):
    x = _mlp3(feat_ref[...], (nw0, nb0, nw1, nb1, nw2, nb2, ng, nbeta))
    e = _mlp3(attr_ref[...], (ew0, eb0, ew1, eb1, ew2, eb2, eg, ebeta))
    h = jnp.maximum(_bdot(x, p1a) + _bdot(e, p1c) + ydst_ref[...] + pb1[...], 0.0)
    h = jnp.maximum(_bdot(h, p2) + pb2[...], 0.0)
    e_upd = _ln(_bdot(h, p3w) + pb3[...], pg[...], pbeta[...]) + e
    out_ref[...] = e_upd.T


def _edges(featp, attrp, y_dst, ne, ee, pw):
    grid = (PE // BM,)
    args = (featp, attrp, y_dst, *ne, *ee, *pw)
    in_specs = [
        _rows(BM, featp.shape[1]),
        _rows(BM, attrp.shape[1]),
        _rows(BM, D),
    ] + [_full(a) for a in args[3:]]
    return pl.pallas_call(
        _edges_kernel,
        grid=grid,
        in_specs=in_specs,
        out_specs=pl.BlockSpec((D, BM), lambda i: (0, i)),
        out_shape=jax.ShapeDtypeStruct((D, PE), F32),
    )(*args)


def _lat_kernel(attr_ref, w0, b0, w1, b1, w2, b2, g, beta, out_ref):
    out_ref[...] = _mlp3(attr_ref[...], (w0, b0, w1, b1, w2, b2, g, beta))


def _lat(latp, le):
    grid = (PL // BM,)
    args = (latp, *le)
    in_specs = [_rows(BM, latp.shape[1])] + [_full(a) for a in args[1:]]
    return pl.pallas_call(
        _lat_kernel,
        grid=grid,
        in_specs=in_specs,
        out_specs=_rows(BM, D),
        out_shape=jax.ShapeDtypeStruct((PL, D), F32),
    )(*args)


def _nodeproc_kernel(x_ref, aggt_ref,
                     m1a, m1b, mb1, m2, mb2, m3, mb3, mg, mbeta,
                     out_ref):
    x = x_ref[...]
    aggt = aggt_ref[...]
    agg = (aggt[0] + aggt[1]).T
    h = jnp.maximum(_bdot(x, m1a) + _bdot(agg, m1b) + mb1[...], 0.0)
    h = jnp.maximum(_bdot(h, m2) + mb2[...], 0.0)
    out_ref[...] = _ln(_bdot(h, m3) + mb3[...], mg[...], mbeta[...]) + x


def _nodeproc(x_h3, aggT, nw):
    bm = 256
    grid = (P3 // bm,)
    args = (x_h3, aggT, *nw)
    in_specs = [
        _rows(bm, D),
        pl.BlockSpec((2, D, bm), lambda i: (0, 0, i)),
    ] + [_full(a) for a in args[2:]]
    return pl.pallas_call(
        _nodeproc_kernel,
        grid=grid,
        in_specs=in_specs,
        out_specs=_rows(bm, D),
        out_shape=jax.ShapeDtypeStruct((P3, D), F32),
    )(*args)


def _sc_gather(table, idx2d):
    mesh = plsc.VectorSubcoreMesh(core_axis_name="core", subcore_axis_name="subcore")

    @functools.partial(
        pl.kernel,
        out_type=jax.ShapeDtypeStruct((PE, D), F32),
        mesh=mesh,
        scratch_types=[
            pltpu.VMEM((CHUNKS_PAD, EW), jnp.int32),
            pltpu.VMEM((EW, D), F32),
            pltpu.VMEM((EW, D), F32),
            pltpu.SemaphoreType.DMA,
            pltpu.SemaphoreType.DMA,
        ],
    )
    def k(table_hbm, i_hbm, o_hbm, idx_v, rows0, rows1, sem0, sem1):
        wid = lax.axis_index("subcore") * 2 + lax.axis_index("core")
        pltpu.sync_copy(i_hbm.at[wid], idx_v)
        base = wid * CHUNKS * EW
        bufs = ((rows0, sem0), (rows1, sem1))

        def start(b, j):
            pltpu.async_copy(table_hbm.at[idx_v.at[j]], bufs[b][0], bufs[b][1])

        def finish(b, j):
            pltpu.make_async_copy(table_hbm.at[idx_v.at[j]], bufs[b][0], bufs[b][1]).wait()
            pltpu.sync_copy(bufs[b][0], o_hbm.at[pl.ds(base + j * EW, EW)])

        start(0, 0)

        @pl.loop(0, CHUNKS // 2)
        def _(p):
            j0 = p * 2
            start(1, j0 + 1)
            finish(0, j0)

            @pl.when(j0 + 2 < CHUNKS)
            def _():
                start(0, j0 + 2)

            finish(1, j0 + 1)

        finish(0, CHUNKS - 1)

    return k(table, idx2d)


def _sc_scatter(e_updT, idx1d, zerosT):
    mesh = plsc.VectorSubcoreMesh(core_axis_name="core", subcore_axis_name="subcore")

    @functools.partial(
        pl.kernel,
        out_type=jax.ShapeDtypeStruct((2, D, P3), F32),
        mesh=mesh,
        scratch_types=[
            pltpu.VMEM((16, P3), F32),
            pltpu.VMEM((CH,), jnp.int32),
            pltpu.VMEM((CH,), jnp.int32),
            pltpu.VMEM((16, CH), F32),
            pltpu.VMEM((16, CH), F32),
            pltpu.SemaphoreType.DMA,
            pltpu.SemaphoreType.DMA,
        ],
        compiler_params=_CP,
    )
    def k(x_hbm, i_hbm, z_hbm, o_hbm, acc, idx0, idx1, x0, x1, sem0, sem1):
        c = lax.axis_index("core")
        s = lax.axis_index("subcore")
        pltpu.sync_copy(z_hbm, acc)
        col0 = s * 16
        ebase = c * HALF
        nch = HALF // CH
        bufs = ((idx0, x0, sem0), (idx1, x1, sem1))

        def start(b, t):
            off = ebase + t * CH
            iv, xv, sem = bufs[b]
            pltpu.async_copy(i_hbm.at[pl.ds(off, CH)], iv, sem)
            pltpu.async_copy(x_hbm.at[pl.ds(col0, 16), pl.ds(off, CH)], xv, sem)

        def compute(b, t):
            off = ebase + t * CH
            iv, xv, sem = bufs[b]
            pltpu.make_async_copy(i_hbm.at[pl.ds(off, CH)], iv, sem).wait()
            pltpu.make_async_copy(
                x_hbm.at[pl.ds(col0, 16), pl.ds(off, CH)], xv, sem
            ).wait()

            @pl.loop(0, CH // 16)
            def _(g):
                v_idx = iv[pl.ds(g * 16, 16)]
                for kk in range(16):
                    rowk = jnp.full((16,), kk, jnp.int32)
                    plsc.addupdate_scatter(
                        acc, [rowk, v_idx], xv[kk, pl.ds(g * 16, 16)]
                    )

        start(0, 0)

        @pl.loop(0, nch // 2)
        def _(p):
            t0 = p * 2
            start(1, t0 + 1)
            compute(0, t0)

            @pl.when(t0 + 2 < nch)
            def _():
                start(0, t0 + 2)

            compute(1, t0 + 1)

        pltpu.sync_copy(acc, o_hbm.at[c, pl.ds(col0, 16)])

    return k(e_updT, idx1d, zerosT)


def _prep_mlp(p):
    (w0, b0), (w1, b1), (w2, b2) = p["layers"]
    r = lambda v: v.reshape(1, -1)
    return (
        w0,
        r(b0),
        w1,
        r(b1),
        w2,
        r(b2),
        r(p["ln_g"]),
        r(p["ln_b"]),
    )


def kernel(features, h3_nodes, in_edge_index, in_edge_attr, lat_edge_index, lat_edge_attr, params):
    n_obs = features.shape[1]
    n3 = h3_nodes.shape[0]
    n_lat = lat_edge_attr.shape[0]

    ne = _prep_mlp(params["node_encoder"])
    ee = _prep_mlp(params["edge_encoder"])
    le = _prep_mlp(params["latent_edge_encoder"])

    pe = params["edge_proc"]
    p1 = pe["layers"][0][0]
    p1a, p1b, p1c = p1[:D], p1[D : 2 * D], p1[2 * D :]
    r = lambda v: v.reshape(1, -1)
    pw = (
        p1a, p1c, r(pe["layers"][0][1]),
        pe["layers"][1][0], r(pe["layers"][1][1]),
        pe["layers"][2][0], r(pe["layers"][2][1]),
        r(pe["ln_g"]), r(pe["ln_b"]),
    )

    pn = params["node_proc"]
    m1 = pn["layers"][0][0]
    nw = (
        m1[:D], m1[D:], r(pn["layers"][0][1]),
        pn["layers"][1][0], r(pn["layers"][1][1]),
        pn["layers"][2][0], r(pn["layers"][2][1]),
        r(pn["ln_g"]), r(pn["ln_b"]),
    )

    featp = jnp.pad(features.reshape(-1, features.shape[-1]), ((0, PE - n_obs), (0, 0)))
    attrp = jnp.pad(in_edge_attr, ((0, PE - n_obs), (0, 0)))
    h3p = jnp.pad(h3_nodes, ((0, P3 - n3), (0, 0)))
    latp = jnp.pad(lat_edge_attr, ((0, PL - n_lat), (0, 0)))
    idx = in_edge_index[1] - n_obs
    pad_tail = n3 + jnp.arange(PE - n_obs, dtype=jnp.int32) % (P3 - n3)
    idxp = jnp.concatenate([idx, pad_tail])
    idx3d_g = jnp.pad(
        idxp.reshape(NW, CHUNKS, EW),
        ((0, 0), (0, CHUNKS_PAD - CHUNKS), (0, 0)),
        constant_values=n3,
    )
    zerosT = jnp.zeros((16, P3), F32)

    x_h3, y_h3 = _h3_prep(h3p, ne, p1b)
    y_dst = _sc_gather(y_h3, idx3d_g)
    e_updT = _edges(featp, attrp, y_dst, ne, ee, pw)
    aggT = _sc_scatter(e_updT, idxp, zerosT)
    lat_e = _lat(latp, le)
    out = _nodeproc(x_h3, aggT, nw)

    return out[:n3], lat_edge_index, lat_e[:n_lat]


# speedup vs baseline: 2.6613x; 1.0952x over previous
import dataclasses
import functools

import jax
import jax.numpy as jnp
from jax import lax
from jax.experimental import pallas as pl
from jax.experimental.pallas import tpu as pltpu
from jax.experimental.pallas import tpu_sc as plsc

F32 = jnp.float32
BF16 = jnp.bfloat16

D = 256
NW = 32
EW = 128
CHUNKS = 13
CHUNKS_PAD = 16
PE = NW * CHUNKS * EW
HALF = PE // 2
CH = 1024
P3 = 5888
PL = 41984
BM = 1024
BM3 = 1472

_CP = pltpu.CompilerParams()
if "needs_layout_passes" in pltpu.CompilerParams.__dataclass_fields__:
    _CP = dataclasses.replace(_CP, needs_layout_passes=False)


def _ln(x, g, b):
    mu = jnp.mean(x, axis=-1, keepdims=True)
    xc = x - mu
    var = jnp.mean(xc * xc, axis=-1, keepdims=True)
    return xc * lax.rsqrt(var + 1e-5) * g + b


def _bdot(a, w_ref):
    return jnp.dot(a, w_ref[...], preferred_element_type=F32)


def _enc_small_k(inp, w0_ref, b0_ref):
    return jnp.maximum(_bdot(inp, w0_ref) + b0_ref[...], 0.0)


def _mlp3(inp, refs):
    w0, b0, w1, b1, w2, b2, g, beta = refs
    h = _enc_small_k(inp, w0, b0)
    h = jnp.maximum(_bdot(h, w1) + b1[...], 0.0)
    return _ln(_bdot(h, w2) + b2[...], g[...], beta[...])


def _full(a):
    return pl.BlockSpec(a.shape, lambda *_: (0,) * a.ndim)


def _rows(bm, nc):
    return pl.BlockSpec((bm, nc), lambda i: (i, 0))


def _h3_prep_kernel(h3_ref, w0, b0, w1, b1, w2, b2, g, beta, p1b, x_out, y_out):
    x = _mlp3(h3_ref[...], (w0, b0, w1, b1, w2, b2, g, beta))
    x_out[...] = x
    y_out[...] = _bdot(x, p1b)


def _h3_prep(h3p, ne, p1b):
    grid = (P3 // BM3,)
    args = (h3p, *ne, p1b)
    in_specs = [_rows(BM3, h3p.shape[1])] + [_full(a) for a in args[1:]]
    return pl.pallas_call(
        _h3_prep_kernel,
        grid=grid,
        in_specs=in_specs,
        out_specs=[_rows(BM3, D), _rows(BM3, D)],
        out_shape=[
            jax.ShapeDtypeStruct((P3, D), F32),
            jax.ShapeDtypeStruct((P3, D), F32),
        ],
    )(*args)


def _edges_kernel(feat_ref, attr_ref, ydst_ref,
                  nw0, nb0, nw1, nb1, nw2, nb2, ng, nbeta,
                  ew0, eb0, ew1, eb1, ew2, eb2, eg, ebeta,
                  p1a, p1c, pb1, p2, pb2, p3w, pb3, pg, pbeta,
                  out_ref):
    x = _mlp3(feat_ref[...], (nw0, nb0, nw1, nb1, nw2, nb2, ng, nbeta))
    e = _mlp3(attr_ref[...], (ew0, eb0, ew1, eb1, ew2, eb2, eg, ebeta))
    h = jnp.maximum(_bdot(x, p1a) + _bdot(e, p1c) + ydst_ref[...] + pb1[...], 0.0)
    h = jnp.maximum(_bdot(h, p2) + pb2[...], 0.0)
    e_upd = _ln(_bdot(h, p3w) + pb3[...], pg[...], pbeta[...]) + e
    out_ref[...] = e_upd.T


def _edges(featp, attrp, y_dst, ne, ee, pw):
    grid = (PE // BM,)
    args = (featp, attrp, y_dst, *ne, *ee, *pw)
    in_specs = [
        _rows(BM, featp.shape[1]),
        _rows(BM, attrp.shape[1]),
        _rows(BM, D),
    ] + [_full(a) for a in args[3:]]
    return pl.pallas_call(
        _edges_kernel,
        grid=grid,
        in_specs=in_specs,
        out_specs=pl.BlockSpec((D, BM), lambda i: (0, i)),
        out_shape=jax.ShapeDtypeStruct((D, PE), F32),
    )(*args)


def _lat_kernel(attr_ref, w0, b0, w1, b1, w2, b2, g, beta, out_ref):
    out_ref[...] = _mlp3(attr_ref[...], (w0, b0, w1, b1, w2, b2, g, beta))


def _lat(latp, le):
    grid = (PL // BM,)
    args = (latp, *le)
    in_specs = [_rows(BM, latp.shape[1])] + [_full(a) for a in args[1:]]
    return pl.pallas_call(
        _lat_kernel,
        grid=grid,
        in_specs=in_specs,
        out_specs=_rows(BM, D),
        out_shape=jax.ShapeDtypeStruct((PL, D), F32),
    )(*args)


def _nodeproc_kernel(x_ref, aggt_ref,
                     m1a, m1b, mb1, m2, mb2, m3, mb3, mg, mbeta,
                     out_ref):
    x = x_ref[...]
    aggt = aggt_ref[...]
    agg = (aggt[0] + aggt[1]).T
    h = jnp.maximum(_bdot(x, m1a) + _bdot(agg, m1b) + mb1[...], 0.0)
    h = jnp.maximum(_bdot(h, m2) + mb2[...], 0.0)
    out_ref[...] = _ln(_bdot(h, m3) + mb3[...], mg[...], mbeta[...]) + x


def _nodeproc(x_h3, aggT, nw):
    bm = 256
    grid = (P3 // bm,)
    args = (x_h3, aggT, *nw)
    in_specs = [
        _rows(bm, D),
        pl.BlockSpec((2, D, bm), lambda i: (0, 0, i)),
    ] + [_full(a) for a in args[2:]]
    return pl.pallas_call(
        _nodeproc_kernel,
        grid=grid,
        in_specs=in_specs,
        out_specs=_rows(bm, D),
        out_shape=jax.ShapeDtypeStruct((P3, D), F32),
    )(*args)


def _sc_gather(table, idx2d):
    mesh = plsc.VectorSubcoreMesh(core_axis_name="core", subcore_axis_name="subcore")

    @functools.partial(
        pl.kernel,
        out_type=jax.ShapeDtypeStruct((PE, D), F32),
        mesh=mesh,
        scratch_types=[
            pltpu.VMEM((CHUNKS_PAD, EW), jnp.int32),
            pltpu.VMEM((EW, D), F32),
            pltpu.VMEM((EW, D), F32),
            pltpu.SemaphoreType.DMA,
            pltpu.SemaphoreType.DMA,
        ],
    )
    def k(table_hbm, i_hbm, o_hbm, idx_v, rows0, rows1, sem0, sem1):
        wid = lax.axis_index("subcore") * 2 + lax.axis_index("core")
        pltpu.sync_copy(i_hbm.at[wid], idx_v)
        base = wid * CHUNKS * EW
        bufs = ((rows0, sem0), (rows1, sem1))

        def start(b, j):
            pltpu.async_copy(table_hbm.at[idx_v.at[j]], bufs[b][0], bufs[b][1])

        def finish(b, j):
            pltpu.make_async_copy(table_hbm.at[idx_v.at[j]], bufs[b][0], bufs[b][1]).wait()
            pltpu.sync_copy(bufs[b][0], o_hbm.at[pl.ds(base + j * EW, EW)])

        start(0, 0)

        @pl.loop(0, CHUNKS // 2)
        def _(p):
            j0 = p * 2
            start(1, j0 + 1)
            finish(0, j0)

            @pl.when(j0 + 2 < CHUNKS)
            def _():
                start(0, j0 + 2)

            finish(1, j0 + 1)

        finish(0, CHUNKS - 1)

    return k(table, idx2d)


def _sc_scatter(e_updT, idx1d, zerosT):
    mesh = plsc.VectorSubcoreMesh(core_axis_name="core", subcore_axis_name="subcore")

    @functools.partial(
        pl.kernel,
        out_type=jax.ShapeDtypeStruct((2, D, P3), F32),
        mesh=mesh,
        scratch_types=[
            pltpu.VMEM((16, P3), F32),
            pltpu.VMEM((CH,), jnp.int32),
            pltpu.VMEM((CH,), jnp.int32),
            pltpu.VMEM((16, CH), F32),
            pltpu.VMEM((16, CH), F32),
            pltpu.SemaphoreType.DMA,
            pltpu.SemaphoreType.DMA,
        ],
        compiler_params=_CP,
    )
    def k(x_hbm, i_hbm, z_hbm, o_hbm, acc, idx0, idx1, x0, x1, sem0, sem1):
        c = lax.axis_index("core")
        s = lax.axis_index("subcore")
        pltpu.sync_copy(z_hbm, acc)
        col0 = s * 16
        ebase = c * HALF
        nch = HALF // CH
        bufs = ((idx0, x0, sem0), (idx1, x1, sem1))

        def start(b, t):
            off = ebase + t * CH
            iv, xv, sem = bufs[b]
            pltpu.async_copy(i_hbm.at[pl.ds(off, CH)], iv, sem)
            pltpu.async_copy(x_hbm.at[pl.ds(col0, 16), pl.ds(off, CH)], xv, sem)

        def compute(b, t):
            off = ebase + t * CH
            iv, xv, sem = bufs[b]
            pltpu.make_async_copy(i_hbm.at[pl.ds(off, CH)], iv, sem).wait()
            pltpu.make_async_copy(
                x_hbm.at[pl.ds(col0, 16), pl.ds(off, CH)], xv, sem
            ).wait()

            @pl.loop(0, CH // 16)
            def _(g):
                v_idx = iv[pl.ds(g * 16, 16)]
                for kk in range(16):
                    rowk = jnp.full((16,), kk, jnp.int32)
                    plsc.addupdate_scatter(
                        acc, [rowk, v_idx], xv[kk, pl.ds(g * 16, 16)]
                    )

        start(0, 0)

        @pl.loop(0, nch // 2)
        def _(p):
            t0 = p * 2
            start(1, t0 + 1)
            compute(0, t0)

            @pl.when(t0 + 2 < nch)
            def _():
                start(0, t0 + 2)

            compute(1, t0 + 1)

        pltpu.sync_copy(acc, o_hbm.at[c, pl.ds(col0, 16)])

    return k(e_updT, idx1d, zerosT)


def _prep_mlp(p):
    (w0, b0), (w1, b1), (w2, b2) = p["layers"]
    r = lambda v: v.reshape(1, -1)
    return (
        w0,
        r(b0),
        w1,
        r(b1),
        w2,
        r(b2),
        r(p["ln_g"]),
        r(p["ln_b"]),
    )


def kernel(features, h3_nodes, in_edge_index, in_edge_attr, lat_edge_index, lat_edge_attr, params):
    n_obs = features.shape[1]
    n3 = h3_nodes.shape[0]
    n_lat = lat_edge_attr.shape[0]

    ne = _prep_mlp(params["node_encoder"])
    ee = _prep_mlp(params["edge_encoder"])
    le = _prep_mlp(params["latent_edge_encoder"])

    pe = params["edge_proc"]
    p1 = pe["layers"][0][0]
    p1a, p1b, p1c = p1[:D], p1[D : 2 * D], p1[2 * D :]
    r = lambda v: v.reshape(1, -1)
    pw = (
        p1a, p1c, r(pe["layers"][0][1]),
        pe["layers"][1][0], r(pe["layers"][1][1]),
        pe["layers"][2][0], r(pe["layers"][2][1]),
        r(pe["ln_g"]), r(pe["ln_b"]),
    )

    pn = params["node_proc"]
    m1 = pn["layers"][0][0]
    nw = (
        m1[:D], m1[D:], r(pn["layers"][0][1]),
        pn["layers"][1][0], r(pn["layers"][1][1]),
        pn["layers"][2][0], r(pn["layers"][2][1]),
        r(pn["ln_g"]), r(pn["ln_b"]),
    )

    featp = jnp.pad(features.reshape(-1, features.shape[-1]), ((0, PE - n_obs), (0, 0)))
    attrp = jnp.pad(in_edge_attr, ((0, PE - n_obs), (0, 0)))
    h3p = jnp.pad(h3_nodes, ((0, P3 - n3), (0, 0)))
    latp = jnp.pad(lat_edge_attr, ((0, PL - n_lat), (0, 0)))
    idx = in_edge_index[1] - n_obs
    pad_tail = n3 + jnp.arange(PE - n_obs, dtype=jnp.int32) % (P3 - n3)
    idxp = jnp.concatenate([idx, pad_tail])
    idx3d_g = jnp.pad(
        idxp.reshape(NW, CHUNKS, EW),
        ((0, 0), (0, CHUNKS_PAD - CHUNKS), (0, 0)),
        constant_values=n3,
    )
    zerosT = jnp.zeros((16, P3), F32)

    x_h3, y_h3 = _h3_prep(h3p, ne, p1b)
    y_dst = _sc_gather(y_h3, idx3d_g)
    e_updT = _edges(featp, attrp, y_dst, ne, ee, pw)
    aggT = _sc_scatter(e_updT, idxp, zerosT)
    lat_e = _lat(latp, le)
    out = _nodeproc(x_h3, aggT, nw)

    return out[:n3], lat_edge_index, lat_e[:n_lat]
